# baseline (speedup 1.0000x reference)
import functools
import math

import jax
import jax.numpy as jnp
from jax.experimental import pallas as pl
from jax.experimental.pallas import tpu as pltpu
from jax.experimental.pallas import tpu_sc as plsc

N = 2048
D = 1024
H = 2048
E = 8
O = 1024
TOPK = 2

TB = 256
NB = N // TB
BLKS = 256
S = N * TOPK + E * BLKS
NBLK = S // BLKS


def _ln(x, g, b):
    m = jnp.mean(x, axis=-1, keepdims=True)
    v = jnp.mean((x - m) ** 2, axis=-1, keepdims=True)
    return (x - m) / jnp.sqrt(v + 1e-5) * g + b


def _dot_t(a, w):
    return jax.lax.dot_general(a.astype(jnp.bfloat16), w.astype(jnp.bfloat16),
                               (((1,), (1,)), ((), ())),
                               preferred_element_type=jnp.float32)


def _gating_body(x_ref, gW0_ref, gb0_ref, gln1_g_ref, gln1_b_ref, gW1_ref, gb1_ref,
                 gln2_g_ref, gln2_b_ref, gW2_ref, gb2_ref, gW3_ref, gb3_ref, temp_ref,
                 probs_ref, idx_ref, topw_ref, lb_ref, usage_ref):
    i = pl.program_id(0)
    x = x_ref[...]
    h0 = jax.nn.relu(_dot_t(x, gW0_ref[...]) + gb0_ref[0])
    h1 = _dot_t(jax.nn.relu(_ln(h0, gln1_g_ref[0], gln1_b_ref[0])), gW1_ref[...]) + gb1_ref[0]
    h1 = h1 + h0
    h2 = _dot_t(jax.nn.relu(_ln(h1, gln2_g_ref[0], gln2_b_ref[0])), gW2_ref[...]) + gb2_ref[0]
    logits = (_dot_t(h2, gW3_ref[...]) + gb3_ref[0]) / temp_ref[0, 0]
    logits = logits - jnp.max(logits, axis=-1, keepdims=True)
    ex = jnp.exp(logits)
    probs = ex / jnp.sum(ex, axis=-1, keepdims=True)
    probs_ref[...] = probs

    lanes = jax.lax.broadcasted_iota(jnp.int32, probs.shape, 1)
    p1 = jnp.max(probs, axis=-1, keepdims=True)
    a1 = jnp.argmax(probs, axis=-1).reshape(-1, 1).astype(jnp.int32)
    masked = jnp.where(lanes == a1, -jnp.inf, probs)
    p2 = jnp.max(masked, axis=-1, keepdims=True)
    a2 = jnp.argmax(masked, axis=-1).reshape(-1, 1).astype(jnp.int32)
    s = p1 + p2
    idx_ref[...] = jnp.concatenate([a1, a2], axis=1)
    topw_ref[...] = jnp.concatenate([p1 / s, p2 / s], axis=1)

    @pl.when(i == 0)
    def _():
        usage_ref[...] = jnp.zeros_like(usage_ref)

    usage_ref[...] += jnp.sum(probs, axis=0, keepdims=True) * (1.0 / N)

    @pl.when(i == pl.num_programs(0) - 1)
    def _():
        usage = usage_ref[0, :]
        ideal = 1.0 / E
        kl = jnp.sum(ideal * (math.log(ideal) - jnp.log(usage + 1e-8)))
        var_loss = jnp.sum((usage - ideal) ** 2)
        ent = -jnp.sum(usage * jnp.log(usage + 1e-8))
        ent_loss = 1.0 - ent / math.log(E)
        lb_ref[...] = jnp.reshape((0.5 * kl + 0.3 * var_loss + 0.2 * ent_loss) * 0.05, (1, 1))


def _gating(x, gW0, gb0, gln1_g, gln1_b, gW1, gb1, gln2_g, gln2_b, gW2, gb2, gW3, gb3,
            temp, interpret=False):
    whole = lambda shape: pl.BlockSpec(shape, lambda i: (0,) * len(shape))
    out_shapes = (
        jax.ShapeDtypeStruct((N, E), jnp.float32),
        jax.ShapeDtypeStruct((N, TOPK), jnp.int32),
        jax.ShapeDtypeStruct((N, TOPK), jnp.float32),
        jax.ShapeDtypeStruct((1, 1), jnp.float32),
    )
    return pl.pallas_call(
        _gating_body,
        grid=(NB,),
        in_specs=[
            pl.BlockSpec((TB, D), lambda i: (i, 0)),
            whole((256, D)), whole((1, 256)), whole((1, 256)), whole((1, 256)),
            whole((256, 256)), whole((1, 256)), whole((1, 256)), whole((1, 256)),
            whole((128, 256)), whole((1, 128)),
            whole((E, 128)), whole((1, E)), whole((1, 1)),
        ],
        out_specs=[
            pl.BlockSpec((TB, E), lambda i: (i, 0)),
            pl.BlockSpec((TB, TOPK), lambda i: (i, 0)),
            pl.BlockSpec((TB, TOPK), lambda i: (i, 0)),
            pl.BlockSpec((1, 1), lambda i: (0, 0)),
        ],
        out_shape=out_shapes,
        scratch_shapes=[pltpu.VMEM((1, E), jnp.float32)],
        interpret=interpret,
    )(x, gW0, gb0.reshape(1, -1), gln1_g.reshape(1, -1), gln1_b.reshape(1, -1),
      gW1, gb1.reshape(1, -1), gln2_g.reshape(1, -1), gln2_b.reshape(1, -1),
      gW2, gb2.reshape(1, -1), gW3, gb3.reshape(1, -1), temp.reshape(1, 1))


def _routing(idx, topw):
    a1, a2 = idx[:, 0], idx[:, 1]
    eids = jnp.concatenate([a1, a2])
    toks = jnp.concatenate([jnp.arange(N, dtype=jnp.int32)] * 2)
    ws = jnp.concatenate([topw[:, 0], topw[:, 1]])
    onehot = (eids[:, None] == jnp.arange(E, dtype=jnp.int32)[None, :]).astype(jnp.int32)
    csum = jnp.cumsum(onehot, axis=0)
    counts = csum[-1]
    rank = jnp.take_along_axis(csum, eids[:, None], axis=1)[:, 0] - 1
    padded = ((counts + BLKS - 1) // BLKS) * BLKS
    poff = jnp.concatenate([jnp.zeros((1,), jnp.int32), jnp.cumsum(padded)[:-1]])
    dest = poff[eids] + rank
    tok_sorted = jnp.zeros((S,), jnp.int32).at[dest].set(toks)
    wsort = jnp.zeros((S, 1), jnp.float32).at[dest, 0].set(ws)
    nblocks = jnp.sum(padded) // BLKS
    bidx = jnp.arange(NBLK, dtype=jnp.int32)
    raw = jnp.sum((bidx[:, None] >= (poff // BLKS)[None, :]).astype(jnp.int32), axis=1) - 1
    block_eid = jnp.take(raw, jnp.minimum(bidx, nblocks - 1))
    return tok_sorted, wsort, dest[:N], dest[N:], block_eid.astype(jnp.int32), \
        jnp.reshape(nblocks, (1,)).astype(jnp.int32)


_NW = 32


def _sc_gather_rows(data, idx, name="gather", chunk=32, nbuf=3):
    M = idx.shape[0]
    C = data.shape[1]
    per_w = M // _NW
    nc = per_w // chunk
    assert M % _NW == 0 and per_w % chunk == 0 and per_w % 8 == 0
    mesh = plsc.VectorSubcoreMesh(core_axis_name="c", subcore_axis_name="s")

    def k(data_hbm, idx_hbm, out_hbm, *scratch):
        idx_v = scratch[0]
        bufs = scratch[1:1 + nbuf]
        gsem = scratch[1 + nbuf:1 + 2 * nbuf]
        ssem = scratch[1 + 2 * nbuf:1 + 3 * nbuf]
        wid = jax.lax.axis_index("s") * 2 + jax.lax.axis_index("c")
        base = wid * per_w
        pltpu.sync_copy(idx_hbm.at[pl.ds(base, per_w)], idx_v)

        def gath(c):
            return pltpu.async_copy(
                data_hbm.at[idx_v.at[pl.ds(c * chunk, chunk)]], bufs[c % nbuf],
                gsem[c % nbuf])

        handles = {c: gath(c) for c in range(min(nbuf, nc))}
        stores = {}
        for c in range(nc):
            handles[c].wait()
            stores[c] = pltpu.async_copy(
                bufs[c % nbuf], out_hbm.at[pl.ds(base + c * chunk, chunk)],
                ssem[c % nbuf])
            nxt = c + nbuf
            if nxt < nc:
                stores[c].wait()
                handles[nxt] = gath(nxt)
        for c in range(max(0, nc - nbuf), nc):
            if c in stores and (c + nbuf >= nc):
                stores[c].wait()

    k.__name__ = name
    kk = pl.kernel(
        k, mesh=mesh,
        out_type=jax.ShapeDtypeStruct((M, C), data.dtype),
        scratch_types=[
            pltpu.VMEM((per_w,), jnp.int32),
            *[pltpu.VMEM((chunk, C), data.dtype) for _ in range(nbuf)],
            *[pltpu.SemaphoreType.DMA for _ in range(2 * nbuf)],
        ],
    )
    return kk(data, idx)


def _k1s_body(beid_ref, nb_ref, xs_ref, W0_ref, b0_ref, ln1_g_ref, ln1_b_ref,
              W1_ref, b1_ref, eh1_ref):
    b = pl.program_id(0)
    e = beid_ref[b]

    @pl.when(b < nb_ref[0])
    def _():
        x = xs_ref[...]
        h0 = jax.nn.relu(_dot_t(x, W0_ref[0]) + b0_ref[e])
        t = _dot_t(jax.nn.relu(_ln(h0, ln1_g_ref[e], ln1_b_ref[e])), W1_ref[0]) + b1_ref[e]
        eh1_ref[...] = t + h0


def _k2s_body(beid_ref, nb_ref, eh1_ref, ln2_g_ref, ln2_b_ref, W2_ref, b2_ref,
              W3_ref, b3_ref, w_ref, ys_ref):
    b = pl.program_id(0)
    e = beid_ref[b]

    @pl.when(b < nb_ref[0])
    def _():
        eh1 = eh1_ref[...]
        eh2 = _dot_t(jax.nn.silu(_ln(eh1, ln2_g_ref[e], ln2_b_ref[e])), W2_ref[0]) + b2_ref[e]
        eout = _dot_t(eh2, W3_ref[0]) + b3_ref[e]
        ys_ref[...] = w_ref[...] * eout


def _experts_sparse(xs, wsort, block_eid, nblocks, eW0, eb0, eln1_g, eln1_b, eW1,
                    eb1, eln2_g, eln2_b, eW2, eb2, eW3, eb3, interpret=False):
    whole = lambda shape: pl.BlockSpec(shape, lambda b, beid, nb: (0,) * len(shape))
    eh1 = pl.pallas_call(
        _k1s_body,
        grid_spec=pltpu.PrefetchScalarGridSpec(
            num_scalar_prefetch=2,
            grid=(NBLK,),
            in_specs=[
                pl.BlockSpec((BLKS, D), lambda b, beid, nb: (b, 0)),
                pl.BlockSpec((1, H, D), lambda b, beid, nb: (beid[b], 0, 0)),
                whole((E, H)), whole((E, H)), whole((E, H)),
                pl.BlockSpec((1, H, H), lambda b, beid, nb: (beid[b], 0, 0)),
                whole((E, H)),
            ],
            out_specs=pl.BlockSpec((BLKS, H), lambda b, beid, nb: (b, 0)),
        ),
        out_shape=jax.ShapeDtypeStruct((S, H), jnp.float32),
        interpret=interpret,
    )(block_eid, nblocks, xs, eW0, eb0, eln1_g, eln1_b, eW1, eb1)

    ys = pl.pallas_call(
        _k2s_body,
        grid_spec=pltpu.PrefetchScalarGridSpec(
            num_scalar_prefetch=2,
            grid=(NBLK,),
            in_specs=[
                pl.BlockSpec((BLKS, H), lambda b, beid, nb: (b, 0)),
                whole((E, H)), whole((E, H)),
                pl.BlockSpec((1, H // 2, H), lambda b, beid, nb: (beid[b], 0, 0)),
                whole((E, H // 2)),
                pl.BlockSpec((1, O, H // 2), lambda b, beid, nb: (beid[b], 0, 0)),
                whole((E, O)),
                pl.BlockSpec((BLKS, 1), lambda b, beid, nb: (b, 0)),
            ],
            out_specs=pl.BlockSpec((BLKS, O), lambda b, beid, nb: (b, 0)),
        ),
        out_shape=jax.ShapeDtypeStruct((S, O), jnp.float32),
        interpret=interpret,
    )(block_eid, nblocks, eh1, eln2_g, eln2_b, eW2, eb2, eW3, eb3, wsort)
    return ys


def _add_body(c0_ref, c1_ref, o_ref):
    o_ref[...] = c0_ref[...] + c1_ref[...]


def _combine(gathered, interpret=False):
    return pl.pallas_call(
        _add_body,
        grid=(NB,),
        in_specs=[
            pl.BlockSpec((TB, O), lambda i: (i, 0)),
            pl.BlockSpec((TB, O), lambda i: (i + NB, 0)),
        ],
        out_specs=pl.BlockSpec((TB, O), lambda i: (i, 0)),
        out_shape=jax.ShapeDtypeStruct((N, O), jnp.float32),
        interpret=interpret,
    )(gathered, gathered)


def _moe(x, eW0, eb0, eln1_g, eln1_b, eW1, eb1, eln2_g, eln2_b, eW2, eb2, eW3, eb3,
         gW0, gb0, gln1_g, gln1_b, gW1, gb1, gln2_g, gln2_b, gW2, gb2, gW3, gb3, temp,
         interpret=False):
    gate_probs, idx, topw, lb = _gating(x, gW0, gb0, gln1_g, gln1_b, gW1, gb1,
                                        gln2_g, gln2_b, gW2, gb2, gW3, gb3, temp,
                                        interpret=interpret)
    tok_sorted, wsort, pos0, pos1, block_eid, nblocks = _routing(idx, topw)
    eW0b, eW1b = eW0.astype(jnp.bfloat16), eW1.astype(jnp.bfloat16)
    eW2b, eW3b = eW2.astype(jnp.bfloat16), eW3.astype(jnp.bfloat16)
    xs = _sc_gather_rows(x, tok_sorted, name="dispatch_gather", chunk=48, nbuf=2)
    ys = _experts_sparse(xs, wsort, block_eid, nblocks, eW0b, eb0, eln1_g, eln1_b,
                         eW1b, eb1, eln2_g, eln2_b, eW2b, eb2, eW3b, eb3,
                         interpret=interpret)
    gathered = _sc_gather_rows(ys, jnp.concatenate([pos0, pos1]), name="combine_gather")
    final = _combine(gathered, interpret=interpret)
    return final, lb[0, 0], gate_probs


def kernel(x, eW0, eb0, eln1_g, eln1_b, eW1, eb1, eln2_g, eln2_b, eW2, eb2, eW3, eb3,
           gW0, gb0, gln1_g, gln1_b, gW1, gb1, gln2_g, gln2_b, gW2, gb2, gW3, gb3, temp):
    return _moe(x, eW0, eb0, eln1_g, eln1_b, eW1, eb1, eln2_g, eln2_b, eW2, eb2, eW3,
                eb3, gW0, gb0, gln1_g, gln1_b, gW1, gb1, gln2_g, gln2_b, gW2, gb2,
                gW3, gb3, temp)


# speedup vs baseline: 1.6375x; 1.6375x over previous
import functools
import math

import jax
import jax.numpy as jnp
from jax.experimental import pallas as pl
from jax.experimental.pallas import tpu as pltpu
from jax.experimental.pallas import tpu_sc as plsc

N = 2048
D = 1024
H = 2048
E = 8
O = 1024
TOPK = 2

TB = 256
NB = N // TB
BLKS = 256
S = N * TOPK + E * BLKS
NBLK = S // BLKS


def _ln(x, g, b):
    m = jnp.mean(x, axis=-1, keepdims=True)
    v = jnp.mean((x - m) ** 2, axis=-1, keepdims=True)
    return (x - m) / jnp.sqrt(v + 1e-5) * g + b


def _dot_t(a, w):
    return jax.lax.dot_general(a.astype(jnp.bfloat16), w.astype(jnp.bfloat16),
                               (((1,), (1,)), ((), ())),
                               preferred_element_type=jnp.float32)


def _gating_body(x_ref, gW0_ref, gb0_ref, gln1_g_ref, gln1_b_ref, gW1_ref, gb1_ref,
                 gln2_g_ref, gln2_b_ref, gW2_ref, gb2_ref, gW3_ref, gb3_ref, temp_ref,
                 probs_ref, idx_ref, topw_ref, lb_ref, usage_ref):
    i = pl.program_id(0)
    x = x_ref[...]
    h0 = jax.nn.relu(_dot_t(x, gW0_ref[...]) + gb0_ref[0])
    h1 = _dot_t(jax.nn.relu(_ln(h0, gln1_g_ref[0], gln1_b_ref[0])), gW1_ref[...]) + gb1_ref[0]
    h1 = h1 + h0
    h2 = _dot_t(jax.nn.relu(_ln(h1, gln2_g_ref[0], gln2_b_ref[0])), gW2_ref[...]) + gb2_ref[0]
    logits = (_dot_t(h2, gW3_ref[...]) + gb3_ref[0]) / temp_ref[0, 0]
    logits = logits - jnp.max(logits, axis=-1, keepdims=True)
    ex = jnp.exp(logits)
    probs = ex / jnp.sum(ex, axis=-1, keepdims=True)
    probs_ref[...] = probs

    lanes = jax.lax.broadcasted_iota(jnp.int32, probs.shape, 1)
    p1 = jnp.max(probs, axis=-1, keepdims=True)
    a1 = jnp.argmax(probs, axis=-1).reshape(-1, 1).astype(jnp.int32)
    masked = jnp.where(lanes == a1, -jnp.inf, probs)
    p2 = jnp.max(masked, axis=-1, keepdims=True)
    a2 = jnp.argmax(masked, axis=-1).reshape(-1, 1).astype(jnp.int32)
    s = p1 + p2
    idx_ref[...] = jnp.concatenate([a1, a2], axis=1)
    topw_ref[...] = jnp.concatenate([p1 / s, p2 / s], axis=1)

    @pl.when(i == 0)
    def _():
        usage_ref[...] = jnp.zeros_like(usage_ref)

    usage_ref[...] += jnp.sum(probs, axis=0, keepdims=True) * (1.0 / N)

    @pl.when(i == pl.num_programs(0) - 1)
    def _():
        usage = usage_ref[0, :]
        ideal = 1.0 / E
        kl = jnp.sum(ideal * (math.log(ideal) - jnp.log(usage + 1e-8)))
        var_loss = jnp.sum((usage - ideal) ** 2)
        ent = -jnp.sum(usage * jnp.log(usage + 1e-8))
        ent_loss = 1.0 - ent / math.log(E)
        lb_ref[...] = jnp.reshape((0.5 * kl + 0.3 * var_loss + 0.2 * ent_loss) * 0.05, (1, 1))


def _gating(x, gW0, gb0, gln1_g, gln1_b, gW1, gb1, gln2_g, gln2_b, gW2, gb2, gW3, gb3,
            temp, interpret=False):
    whole = lambda shape: pl.BlockSpec(shape, lambda i: (0,) * len(shape))
    out_shapes = (
        jax.ShapeDtypeStruct((N, E), jnp.float32),
        jax.ShapeDtypeStruct((N, TOPK), jnp.int32),
        jax.ShapeDtypeStruct((N, TOPK), jnp.float32),
        jax.ShapeDtypeStruct((1, 1), jnp.float32),
    )
    return pl.pallas_call(
        _gating_body,
        grid=(NB,),
        in_specs=[
            pl.BlockSpec((TB, D), lambda i: (i, 0)),
            whole((256, D)), whole((1, 256)), whole((1, 256)), whole((1, 256)),
            whole((256, 256)), whole((1, 256)), whole((1, 256)), whole((1, 256)),
            whole((128, 256)), whole((1, 128)),
            whole((E, 128)), whole((1, E)), whole((1, 1)),
        ],
        out_specs=[
            pl.BlockSpec((TB, E), lambda i: (i, 0)),
            pl.BlockSpec((TB, TOPK), lambda i: (i, 0)),
            pl.BlockSpec((TB, TOPK), lambda i: (i, 0)),
            pl.BlockSpec((1, 1), lambda i: (0, 0)),
        ],
        out_shape=out_shapes,
        scratch_shapes=[pltpu.VMEM((1, E), jnp.float32)],
        interpret=interpret,
    )(x, gW0, gb0.reshape(1, -1), gln1_g.reshape(1, -1), gln1_b.reshape(1, -1),
      gW1, gb1.reshape(1, -1), gln2_g.reshape(1, -1), gln2_b.reshape(1, -1),
      gW2, gb2.reshape(1, -1), gW3, gb3.reshape(1, -1), temp.reshape(1, 1))


def _routing(idx, topw):
    a1, a2 = idx[:, 0], idx[:, 1]
    eids = jnp.concatenate([a1, a2])
    toks = jnp.concatenate([jnp.arange(N, dtype=jnp.int32)] * 2)
    ws = jnp.concatenate([topw[:, 0], topw[:, 1]])
    onehot = (eids[:, None] == jnp.arange(E, dtype=jnp.int32)[None, :]).astype(jnp.int32)
    csum = jnp.cumsum(onehot, axis=0)
    counts = csum[-1]
    rank = jnp.take_along_axis(csum, eids[:, None], axis=1)[:, 0] - 1
    padded = ((counts + BLKS - 1) // BLKS) * BLKS
    poff = jnp.concatenate([jnp.zeros((1,), jnp.int32), jnp.cumsum(padded)[:-1]])
    dest = poff[eids] + rank
    tok_sorted = (jnp.arange(S, dtype=jnp.int32) % N).at[dest].set(toks)
    wsort = jnp.zeros((S, 1), jnp.float32).at[dest, 0].set(ws)
    nblocks = jnp.sum(padded) // BLKS
    bidx = jnp.arange(NBLK, dtype=jnp.int32)
    raw = jnp.sum((bidx[:, None] >= (poff // BLKS)[None, :]).astype(jnp.int32), axis=1) - 1
    block_eid = jnp.take(raw, jnp.minimum(bidx, nblocks - 1))
    return tok_sorted, wsort, dest[:N], dest[N:], block_eid.astype(jnp.int32), \
        jnp.reshape(nblocks, (1,)).astype(jnp.int32)


_NW = 32


def _sc_gather_rows(data, idx, name="gather", chunk=32, nbuf=3):
    M = idx.shape[0]
    C = data.shape[1]
    per_w = M // _NW
    nc = per_w // chunk
    assert M % _NW == 0 and per_w % chunk == 0 and per_w % 8 == 0
    mesh = plsc.VectorSubcoreMesh(core_axis_name="c", subcore_axis_name="s")

    def k(data_hbm, idx_hbm, out_hbm, *scratch):
        idx_v = scratch[0]
        bufs = scratch[1:1 + nbuf]
        gsem = scratch[1 + nbuf:1 + 2 * nbuf]
        ssem = scratch[1 + 2 * nbuf:1 + 3 * nbuf]
        wid = jax.lax.axis_index("s") * 2 + jax.lax.axis_index("c")
        base = wid * per_w
        pltpu.sync_copy(idx_hbm.at[pl.ds(base, per_w)], idx_v)

        def gath(c):
            return pltpu.async_copy(
                data_hbm.at[idx_v.at[pl.ds(c * chunk, chunk)]], bufs[c % nbuf],
                gsem[c % nbuf])

        handles = {c: gath(c) for c in range(min(nbuf, nc))}
        stores = {}
        for c in range(nc):
            handles[c].wait()
            stores[c] = pltpu.async_copy(
                bufs[c % nbuf], out_hbm.at[pl.ds(base + c * chunk, chunk)],
                ssem[c % nbuf])
            nxt = c + nbuf
            if nxt < nc:
                stores[c].wait()
                handles[nxt] = gath(nxt)
        for c in range(max(0, nc - nbuf), nc):
            if c in stores and (c + nbuf >= nc):
                stores[c].wait()

    k.__name__ = name
    kk = pl.kernel(
        k, mesh=mesh,
        out_type=jax.ShapeDtypeStruct((M, C), data.dtype),
        scratch_types=[
            pltpu.VMEM((per_w,), jnp.int32),
            *[pltpu.VMEM((chunk, C), data.dtype) for _ in range(nbuf)],
            *[pltpu.SemaphoreType.DMA for _ in range(2 * nbuf)],
        ],
    )
    return kk(data, idx)


def _k1s_body(beid_ref, nb_ref, xs_ref, W0_ref, b0_ref, ln1_g_ref, ln1_b_ref,
              W1_ref, b1_ref, eh1_ref):
    b = pl.program_id(0)
    e = beid_ref[b]

    @pl.when(b < nb_ref[0])
    def _():
        x = xs_ref[...]
        h0 = jax.nn.relu(_dot_t(x, W0_ref[0]) + b0_ref[e])
        t = _dot_t(jax.nn.relu(_ln(h0, ln1_g_ref[e], ln1_b_ref[e])), W1_ref[0]) + b1_ref[e]
        eh1_ref[...] = t + h0


def _k2s_body(beid_ref, nb_ref, eh1_ref, ln2_g_ref, ln2_b_ref, W2_ref, b2_ref,
              W3_ref, b3_ref, w_ref, ys_ref):
    b = pl.program_id(0)
    e = beid_ref[b]

    @pl.when(b < nb_ref[0])
    def _():
        eh1 = eh1_ref[...]
        eh2 = _dot_t(jax.nn.silu(_ln(eh1, ln2_g_ref[e], ln2_b_ref[e])), W2_ref[0]) + b2_ref[e]
        eout = _dot_t(eh2, W3_ref[0]) + b3_ref[e]
        ys_ref[...] = w_ref[...] * eout


def _experts_sparse(xs, wsort, block_eid, nblocks, eW0, eb0, eln1_g, eln1_b, eW1,
                    eb1, eln2_g, eln2_b, eW2, eb2, eW3, eb3, interpret=False):
    whole = lambda shape: pl.BlockSpec(shape, lambda b, beid, nb: (0,) * len(shape))
    eh1 = pl.pallas_call(
        _k1s_body,
        grid_spec=pltpu.PrefetchScalarGridSpec(
            num_scalar_prefetch=2,
            grid=(NBLK,),
            in_specs=[
                pl.BlockSpec((BLKS, D), lambda b, beid, nb: (b, 0)),
                pl.BlockSpec((1, H, D), lambda b, beid, nb: (beid[b], 0, 0)),
                whole((E, H)), whole((E, H)), whole((E, H)),
                pl.BlockSpec((1, H, H), lambda b, beid, nb: (beid[b], 0, 0)),
                whole((E, H)),
            ],
            out_specs=pl.BlockSpec((BLKS, H), lambda b, beid, nb: (b, 0)),
        ),
        out_shape=jax.ShapeDtypeStruct((S, H), jnp.float32),
        interpret=interpret,
    )(block_eid, nblocks, xs, eW0, eb0, eln1_g, eln1_b, eW1, eb1)

    ys = pl.pallas_call(
        _k2s_body,
        grid_spec=pltpu.PrefetchScalarGridSpec(
            num_scalar_prefetch=2,
            grid=(NBLK,),
            in_specs=[
                pl.BlockSpec((BLKS, H), lambda b, beid, nb: (b, 0)),
                whole((E, H)), whole((E, H)),
                pl.BlockSpec((1, H // 2, H), lambda b, beid, nb: (beid[b], 0, 0)),
                whole((E, H // 2)),
                pl.BlockSpec((1, O, H // 2), lambda b, beid, nb: (beid[b], 0, 0)),
                whole((E, O)),
                pl.BlockSpec((BLKS, 1), lambda b, beid, nb: (b, 0)),
            ],
            out_specs=pl.BlockSpec((BLKS, O), lambda b, beid, nb: (b, 0)),
        ),
        out_shape=jax.ShapeDtypeStruct((S, O), jnp.float32),
        interpret=interpret,
    )(block_eid, nblocks, eh1, eln2_g, eln2_b, eW2, eb2, eW3, eb3, wsort)
    return ys


def _add_body(c0_ref, c1_ref, o_ref):
    o_ref[...] = c0_ref[...] + c1_ref[...]


def _combine(gathered, interpret=False):
    return pl.pallas_call(
        _add_body,
        grid=(NB,),
        in_specs=[
            pl.BlockSpec((TB, O), lambda i: (i, 0)),
            pl.BlockSpec((TB, O), lambda i: (i + NB, 0)),
        ],
        out_specs=pl.BlockSpec((TB, O), lambda i: (i, 0)),
        out_shape=jax.ShapeDtypeStruct((N, O), jnp.float32),
        interpret=interpret,
    )(gathered, gathered)


def _moe(x, eW0, eb0, eln1_g, eln1_b, eW1, eb1, eln2_g, eln2_b, eW2, eb2, eW3, eb3,
         gW0, gb0, gln1_g, gln1_b, gW1, gb1, gln2_g, gln2_b, gW2, gb2, gW3, gb3, temp,
         interpret=False):
    gate_probs, idx, topw, lb = _gating(x, gW0, gb0, gln1_g, gln1_b, gW1, gb1,
                                        gln2_g, gln2_b, gW2, gb2, gW3, gb3, temp,
                                        interpret=interpret)
    tok_sorted, wsort, pos0, pos1, block_eid, nblocks = _routing(idx, topw)
    xs = _sc_gather_rows(x, tok_sorted, name="dispatch_gather")
    ys = _experts_sparse(xs, wsort, block_eid, nblocks, eW0, eb0, eln1_g, eln1_b,
                         eW1, eb1, eln2_g, eln2_b, eW2, eb2, eW3, eb3,
                         interpret=interpret)
    gathered = _sc_gather_rows(ys, jnp.concatenate([pos0, pos1]), name="combine_gather")
    final = _combine(gathered, interpret=interpret)
    return final, lb[0, 0], gate_probs


def kernel(x, eW0, eb0, eln1_g, eln1_b, eW1, eb1, eln2_g, eln2_b, eW2, eb2, eW3, eb3,
           gW0, gb0, gln1_g, gln1_b, gW1, gb1, gln2_g, gln2_b, gW2, gb2, gW3, gb3, temp):
    return _moe(x, eW0, eb0, eln1_g, eln1_b, eW1, eb1, eln2_g, eln2_b, eW2, eb2, eW3,
                eb3, gW0, gb0, gln1_g, gln1_b, gW1, gb1, gln2_g, gln2_b, gW2, gb2,
                gW3, gb3, temp)


# speedup vs baseline: 1.6748x; 1.0228x over previous
import functools
import math

import jax
import jax.numpy as jnp
from jax.experimental import pallas as pl
from jax.experimental.pallas import tpu as pltpu
from jax.experimental.pallas import tpu_sc as plsc

N = 2048
D = 1024
H = 2048
E = 8
O = 1024
TOPK = 2

TB = 256
NB = N // TB
BLKS = 256
S = N * TOPK + E * BLKS
NBLK = S // BLKS


def _ln(x, g, b):
    m = jnp.mean(x, axis=-1, keepdims=True)
    v = jnp.mean((x - m) ** 2, axis=-1, keepdims=True)
    return (x - m) / jnp.sqrt(v + 1e-5) * g + b


def _dot_t(a, w):
    return jax.lax.dot_general(a.astype(jnp.bfloat16), w.astype(jnp.bfloat16),
                               (((1,), (1,)), ((), ())),
                               preferred_element_type=jnp.float32)


def _gating_body(x_ref, gW0_ref, gb0_ref, gln1_g_ref, gln1_b_ref, gW1_ref, gb1_ref,
                 gln2_g_ref, gln2_b_ref, gW2_ref, gb2_ref, gW3_ref, gb3_ref, temp_ref,
                 probs_ref, idx_ref, topw_ref, lb_ref, usage_ref):
    i = pl.program_id(0)
    x = x_ref[...]
    h0 = jax.nn.relu(_dot_t(x, gW0_ref[...]) + gb0_ref[0])
    h1 = _dot_t(jax.nn.relu(_ln(h0, gln1_g_ref[0], gln1_b_ref[0])), gW1_ref[...]) + gb1_ref[0]
    h1 = h1 + h0
    h2 = _dot_t(jax.nn.relu(_ln(h1, gln2_g_ref[0], gln2_b_ref[0])), gW2_ref[...]) + gb2_ref[0]
    logits = (_dot_t(h2, gW3_ref[...]) + gb3_ref[0]) / temp_ref[0, 0]
    logits = logits - jnp.max(logits, axis=-1, keepdims=True)
    ex = jnp.exp(logits)
    probs = ex / jnp.sum(ex, axis=-1, keepdims=True)
    probs_ref[...] = probs

    lanes = jax.lax.broadcasted_iota(jnp.int32, probs.shape, 1)
    p1 = jnp.max(probs, axis=-1, keepdims=True)
    a1 = jnp.argmax(probs, axis=-1).reshape(-1, 1).astype(jnp.int32)
    masked = jnp.where(lanes == a1, -jnp.inf, probs)
    p2 = jnp.max(masked, axis=-1, keepdims=True)
    a2 = jnp.argmax(masked, axis=-1).reshape(-1, 1).astype(jnp.int32)
    s = p1 + p2
    idx_ref[...] = jnp.concatenate([a1, a2], axis=1)
    topw_ref[...] = jnp.concatenate([p1 / s, p2 / s], axis=1)

    @pl.when(i == 0)
    def _():
        usage_ref[...] = jnp.zeros_like(usage_ref)

    usage_ref[...] += jnp.sum(probs, axis=0, keepdims=True) * (1.0 / N)

    @pl.when(i == pl.num_programs(0) - 1)
    def _():
        usage = usage_ref[0, :]
        ideal = 1.0 / E
        kl = jnp.sum(ideal * (math.log(ideal) - jnp.log(usage + 1e-8)))
        var_loss = jnp.sum((usage - ideal) ** 2)
        ent = -jnp.sum(usage * jnp.log(usage + 1e-8))
        ent_loss = 1.0 - ent / math.log(E)
        lb_ref[...] = jnp.reshape((0.5 * kl + 0.3 * var_loss + 0.2 * ent_loss) * 0.05, (1, 1))


def _gating(x, gW0, gb0, gln1_g, gln1_b, gW1, gb1, gln2_g, gln2_b, gW2, gb2, gW3, gb3,
            temp, interpret=False):
    whole = lambda shape: pl.BlockSpec(shape, lambda i: (0,) * len(shape))
    out_shapes = (
        jax.ShapeDtypeStruct((N, E), jnp.float32),
        jax.ShapeDtypeStruct((N, TOPK), jnp.int32),
        jax.ShapeDtypeStruct((N, TOPK), jnp.float32),
        jax.ShapeDtypeStruct((1, 1), jnp.float32),
    )
    return pl.pallas_call(
        _gating_body,
        grid=(NB,),
        in_specs=[
            pl.BlockSpec((TB, D), lambda i: (i, 0)),
            whole((256, D)), whole((1, 256)), whole((1, 256)), whole((1, 256)),
            whole((256, 256)), whole((1, 256)), whole((1, 256)), whole((1, 256)),
            whole((128, 256)), whole((1, 128)),
            whole((E, 128)), whole((1, E)), whole((1, 1)),
        ],
        out_specs=[
            pl.BlockSpec((TB, E), lambda i: (i, 0)),
            pl.BlockSpec((TB, TOPK), lambda i: (i, 0)),
            pl.BlockSpec((TB, TOPK), lambda i: (i, 0)),
            pl.BlockSpec((1, 1), lambda i: (0, 0)),
        ],
        out_shape=out_shapes,
        scratch_shapes=[pltpu.VMEM((1, E), jnp.float32)],
        interpret=interpret,
    )(x, gW0, gb0.reshape(1, -1), gln1_g.reshape(1, -1), gln1_b.reshape(1, -1),
      gW1, gb1.reshape(1, -1), gln2_g.reshape(1, -1), gln2_b.reshape(1, -1),
      gW2, gb2.reshape(1, -1), gW3, gb3.reshape(1, -1), temp.reshape(1, 1))


def _routing(idx, topw):
    a1, a2 = idx[:, 0], idx[:, 1]
    eids = jnp.concatenate([a1, a2])
    toks = jnp.concatenate([jnp.arange(N, dtype=jnp.int32)] * 2)
    ws = jnp.concatenate([topw[:, 0], topw[:, 1]])
    onehot = (eids[:, None] == jnp.arange(E, dtype=jnp.int32)[None, :]).astype(jnp.int32)
    csum = jnp.cumsum(onehot, axis=0)
    counts = csum[-1]
    rank = jnp.take_along_axis(csum, eids[:, None], axis=1)[:, 0] - 1
    padded = ((counts + BLKS - 1) // BLKS) * BLKS
    poff = jnp.concatenate([jnp.zeros((1,), jnp.int32), jnp.cumsum(padded)[:-1]])
    dest = poff[eids] + rank
    tok_sorted = (jnp.arange(S, dtype=jnp.int32) % N).at[dest].set(toks)
    wsort = jnp.zeros((S, 1), jnp.float32).at[dest, 0].set(ws)
    nblocks = jnp.sum(padded) // BLKS
    bidx = jnp.arange(NBLK, dtype=jnp.int32)
    raw = jnp.sum((bidx[:, None] >= (poff // BLKS)[None, :]).astype(jnp.int32), axis=1) - 1
    block_eid = jnp.take(raw, jnp.minimum(bidx, nblocks - 1))
    return tok_sorted, wsort, dest[:N], dest[N:], block_eid.astype(jnp.int32), \
        jnp.reshape(nblocks, (1,)).astype(jnp.int32)


_NW = 32


def _sc_gather_rows(data, idx, name="gather", chunk=32, nbuf=3):
    M = idx.shape[0]
    C = data.shape[1]
    per_w = M // _NW
    nc = per_w // chunk
    assert M % _NW == 0 and per_w % chunk == 0 and per_w % 8 == 0
    mesh = plsc.VectorSubcoreMesh(core_axis_name="c", subcore_axis_name="s")

    def k(data_hbm, idx_hbm, out_hbm, *scratch):
        idx_v = scratch[0]
        bufs = scratch[1:1 + nbuf]
        gsem = scratch[1 + nbuf:1 + 2 * nbuf]
        ssem = scratch[1 + 2 * nbuf:1 + 3 * nbuf]
        wid = jax.lax.axis_index("s") * 2 + jax.lax.axis_index("c")
        base = wid * per_w
        pltpu.sync_copy(idx_hbm.at[pl.ds(base, per_w)], idx_v)

        def gath(c):
            return pltpu.async_copy(
                data_hbm.at[idx_v.at[pl.ds(c * chunk, chunk)]], bufs[c % nbuf],
                gsem[c % nbuf])

        handles = {c: gath(c) for c in range(min(nbuf, nc))}
        stores = {}
        for c in range(nc):
            handles[c].wait()
            stores[c] = pltpu.async_copy(
                bufs[c % nbuf], out_hbm.at[pl.ds(base + c * chunk, chunk)],
                ssem[c % nbuf])
            nxt = c + nbuf
            if nxt < nc:
                stores[c].wait()
                handles[nxt] = gath(nxt)
        for c in range(max(0, nc - nbuf), nc):
            if c in stores and (c + nbuf >= nc):
                stores[c].wait()

    k.__name__ = name
    kk = pl.kernel(
        k, mesh=mesh,
        out_type=jax.ShapeDtypeStruct((M, C), data.dtype),
        scratch_types=[
            pltpu.VMEM((per_w,), jnp.int32),
            *[pltpu.VMEM((chunk, C), data.dtype) for _ in range(nbuf)],
            *[pltpu.SemaphoreType.DMA for _ in range(2 * nbuf)],
        ],
    )
    return kk(data, idx)


def _k1s_body(beid_ref, nb_ref, xs_ref, W0_ref, b0_ref, ln1_g_ref, ln1_b_ref,
              W1_ref, b1_ref, eh1_ref):
    b = pl.program_id(0)
    e = beid_ref[b]

    @pl.when(b < nb_ref[0])
    def _():
        x = xs_ref[...]
        h0 = jax.nn.relu(_dot_t(x, W0_ref[0]) + b0_ref[e])
        t = _dot_t(jax.nn.relu(_ln(h0, ln1_g_ref[e], ln1_b_ref[e])), W1_ref[0]) + b1_ref[e]
        eh1_ref[...] = (t + h0).astype(jnp.bfloat16)


def _k2s_body(beid_ref, nb_ref, eh1_ref, ln2_g_ref, ln2_b_ref, W2_ref, b2_ref,
              W3_ref, b3_ref, w_ref, ys_ref):
    b = pl.program_id(0)
    e = beid_ref[b]

    @pl.when(b < nb_ref[0])
    def _():
        eh1 = eh1_ref[...].astype(jnp.float32)
        eh2 = _dot_t(jax.nn.silu(_ln(eh1, ln2_g_ref[e], ln2_b_ref[e])), W2_ref[0]) + b2_ref[e]
        eout = _dot_t(eh2, W3_ref[0]) + b3_ref[e]
        ys_ref[...] = w_ref[...] * eout


def _experts_sparse(xs, wsort, block_eid, nblocks, eW0, eb0, eln1_g, eln1_b, eW1,
                    eb1, eln2_g, eln2_b, eW2, eb2, eW3, eb3, interpret=False):
    whole = lambda shape: pl.BlockSpec(shape, lambda b, beid, nb: (0,) * len(shape))
    eh1 = pl.pallas_call(
        _k1s_body,
        grid_spec=pltpu.PrefetchScalarGridSpec(
            num_scalar_prefetch=2,
            grid=(NBLK,),
            in_specs=[
                pl.BlockSpec((BLKS, D), lambda b, beid, nb: (b, 0)),
                pl.BlockSpec((1, H, D), lambda b, beid, nb: (beid[b], 0, 0)),
                whole((E, H)), whole((E, H)), whole((E, H)),
                pl.BlockSpec((1, H, H), lambda b, beid, nb: (beid[b], 0, 0)),
                whole((E, H)),
            ],
            out_specs=pl.BlockSpec((BLKS, H), lambda b, beid, nb: (b, 0)),
        ),
        out_shape=jax.ShapeDtypeStruct((S, H), jnp.bfloat16),
        interpret=interpret,
    )(block_eid, nblocks, xs, eW0, eb0, eln1_g, eln1_b, eW1, eb1)

    ys = pl.pallas_call(
        _k2s_body,
        grid_spec=pltpu.PrefetchScalarGridSpec(
            num_scalar_prefetch=2,
            grid=(NBLK,),
            in_specs=[
                pl.BlockSpec((BLKS, H), lambda b, beid, nb: (b, 0)),
                whole((E, H)), whole((E, H)),
                pl.BlockSpec((1, H // 2, H), lambda b, beid, nb: (beid[b], 0, 0)),
                whole((E, H // 2)),
                pl.BlockSpec((1, O, H // 2), lambda b, beid, nb: (beid[b], 0, 0)),
                whole((E, O)),
                pl.BlockSpec((BLKS, 1), lambda b, beid, nb: (b, 0)),
            ],
            out_specs=pl.BlockSpec((BLKS, O), lambda b, beid, nb: (b, 0)),
        ),
        out_shape=jax.ShapeDtypeStruct((S, O), jnp.float32),
        interpret=interpret,
    )(block_eid, nblocks, eh1, eln2_g, eln2_b, eW2, eb2, eW3, eb3, wsort)
    return ys


def _add_body(c0_ref, c1_ref, o_ref):
    o_ref[...] = c0_ref[...] + c1_ref[...]


def _combine(gathered, interpret=False):
    return pl.pallas_call(
        _add_body,
        grid=(NB,),
        in_specs=[
            pl.BlockSpec((TB, O), lambda i: (i, 0)),
            pl.BlockSpec((TB, O), lambda i: (i + NB, 0)),
        ],
        out_specs=pl.BlockSpec((TB, O), lambda i: (i, 0)),
        out_shape=jax.ShapeDtypeStruct((N, O), jnp.float32),
        interpret=interpret,
    )(gathered, gathered)


def _moe(x, eW0, eb0, eln1_g, eln1_b, eW1, eb1, eln2_g, eln2_b, eW2, eb2, eW3, eb3,
         gW0, gb0, gln1_g, gln1_b, gW1, gb1, gln2_g, gln2_b, gW2, gb2, gW3, gb3, temp,
         interpret=False):
    gate_probs, idx, topw, lb = _gating(x, gW0, gb0, gln1_g, gln1_b, gW1, gb1,
                                        gln2_g, gln2_b, gW2, gb2, gW3, gb3, temp,
                                        interpret=interpret)
    tok_sorted, wsort, pos0, pos1, block_eid, nblocks = _routing(idx, topw)
    xs = _sc_gather_rows(x, tok_sorted, name="dispatch_gather")
    ys = _experts_sparse(xs, wsort, block_eid, nblocks, eW0, eb0, eln1_g, eln1_b,
                         eW1, eb1, eln2_g, eln2_b, eW2, eb2, eW3, eb3,
                         interpret=interpret)
    gathered = _sc_gather_rows(ys, jnp.concatenate([pos0, pos1]), name="combine_gather")
    final = _combine(gathered, interpret=interpret)
    return final, lb[0, 0], gate_probs


def kernel(x, eW0, eb0, eln1_g, eln1_b, eW1, eb1, eln2_g, eln2_b, eW2, eb2, eW3, eb3,
           gW0, gb0, gln1_g, gln1_b, gW1, gb1, gln2_g, gln2_b, gW2, gb2, gW3, gb3, temp):
    return _moe(x, eW0, eb0, eln1_g, eln1_b, eW1, eb1, eln2_g, eln2_b, eW2, eb2, eW3,
                eb3, gW0, gb0, gln1_g, gln1_b, gW1, gb1, gln2_g, gln2_b, gW2, gb2,
                gW3, gb3, temp)
